# baseline (speedup 1.0000x reference)
import functools

import jax
import jax.numpy as jnp
from jax import lax
from jax.experimental import pallas as pl
from jax.experimental.pallas import tpu as pltpu
from jax.experimental.pallas import tpu_sc as plsc

H, W = 512, 512
HW = H * W
B = 128
NC, NS, LANES = 2, 16, 16
NW = NC * NS
SPW = B // NW
CHUNK = 32768
NCHUNK = HW // CHUNK
INT_MAX = 2**31 - 1


def _rotreduce(v, tmp, op):
    for shift in (8, 4, 2, 1):
        tmp[pl.ds(0, LANES)] = v
        tmp[pl.ds(LANES, LANES)] = v
        v = op(v, tmp[pl.ds(shift, LANES)])
    return v


def _argmax_gather_kernel(inp, gx, gy, outx, outy, buf0, buf1, idxv, gatv,
                          tmpf, tmpi, sem0, sem1, gsem):
    cid = lax.axis_index("c")
    sid = lax.axis_index("s")
    wid = sid * NC + cid
    iota = lax.iota(jnp.int32, LANES)
    bufs = (buf0, buf1)
    sems = (sem0, sem1)

    def chunk_copy(g, buf, sem):
        s = wid * SPW + (g // NCHUNK)
        c = g % NCHUNK
        return pltpu.make_async_copy(
            inp.at[s, pl.ds(c * CHUNK, CHUNK)], buf, sem)

    chunk_copy(0, bufs[0], sems[0]).start()

    idx_lanes = jnp.zeros((LANES,), jnp.int32)
    total = SPW * NCHUNK
    for s_local in range(SPW):
        run_max = jnp.full((LANES,), -jnp.inf, jnp.float32)
        run_idx = jnp.zeros((LANES,), jnp.int32)
        for c in range(NCHUNK):
            g = s_local * NCHUNK + c
            buf, sem = bufs[g % 2], sems[g % 2]
            if g + 1 < total:
                chunk_copy(g + 1, bufs[(g + 1) % 2], sems[(g + 1) % 2]).start()
            chunk_copy(g, buf, sem).wait()

            def body(i, carry, buf=buf, base=c * CHUNK):
                rm, ri = carry
                v = buf[pl.ds(i * LANES, LANES)]
                iv = iota + (base + i * LANES)
                m = v > rm
                return jnp.where(m, v, rm), jnp.where(m, iv, ri)

            run_max, run_idx = lax.fori_loop(
                0, CHUNK // LANES, body, (run_max, run_idx))
        m = _rotreduce(run_max, tmpf, jnp.maximum)
        cand = jnp.where(run_max == m, run_idx, jnp.int32(INT_MAX))
        best = _rotreduce(cand, tmpi, jnp.minimum)
        idx_lanes = jnp.where(iota == s_local, best, idx_lanes)

    idxv[...] = idx_lanes
    pltpu.make_async_copy(gx.at[idxv], gatv, gsem).start()
    pltpu.make_async_copy(gx.at[idxv], gatv, gsem).wait()
    pltpu.sync_copy(gatv, outx.at[wid])
    pltpu.make_async_copy(gy.at[idxv], gatv, gsem).start()
    pltpu.make_async_copy(gy.at[idxv], gatv, gsem).wait()
    pltpu.sync_copy(gatv, outy.at[wid])


@jax.jit
def kernel(input, grid_x, grid_y):
    inp2 = input.reshape(B, HW)
    gx1 = grid_x.reshape(HW)
    gy1 = grid_y.reshape(HW)
    call = functools.partial(
        pl.kernel,
        out_type=[
            jax.ShapeDtypeStruct((NW, LANES), jnp.float32),
            jax.ShapeDtypeStruct((NW, LANES), jnp.float32),
        ],
        mesh=plsc.VectorSubcoreMesh(core_axis_name="c", subcore_axis_name="s"),
        scratch_types=[
            pltpu.VMEM((CHUNK,), jnp.float32),
            pltpu.VMEM((CHUNK,), jnp.float32),
            pltpu.VMEM((LANES,), jnp.int32),
            pltpu.VMEM((LANES,), jnp.float32),
            pltpu.VMEM((2 * LANES,), jnp.float32),
            pltpu.VMEM((2 * LANES,), jnp.int32),
            pltpu.SemaphoreType.DMA,
            pltpu.SemaphoreType.DMA,
            pltpu.SemaphoreType.DMA,
        ],
    )(_argmax_gather_kernel)
    outx, outy = call(inp2, gx1, gy1)
    x = outx[:, :SPW].reshape(B, 1)
    y = outy[:, :SPW].reshape(B, 1)
    return jnp.concatenate((x, y), axis=1)


# speedup vs baseline: 2.3441x; 2.3441x over previous
import functools

import jax
import jax.numpy as jnp
from jax import lax
from jax.experimental import pallas as pl
from jax.experimental.pallas import tpu as pltpu
from jax.experimental.pallas import tpu_sc as plsc

H, W = 512, 512
HW = H * W
B = 128
NC, NS, LANES = 2, 16, 16
NW = NC * NS
SPW = B // NW
CHUNK = 32768
NCHUNK = HW // CHUNK
UNROLL = 8
INT_MAX = 2**31 - 1


def _rotreduce(v, tmp, op):
    for shift in (8, 4, 2, 1):
        tmp[pl.ds(0, LANES)] = v
        tmp[pl.ds(LANES, LANES)] = v
        v = op(v, tmp[pl.ds(shift, LANES)])
    return v


def _argmax_gather_kernel(inp, gx, gy, outx, outy, buf0, buf1, idxv, gatv,
                          tmpf, tmpi, sem0, sem1, gsem):
    cid = lax.axis_index("c")
    sid = lax.axis_index("s")
    wid = sid * NC + cid
    iota = lax.iota(jnp.int32, LANES)
    bufs = (buf0, buf1)
    sems = (sem0, sem1)

    def chunk_copy(g, buf, sem):
        s = wid * SPW + (g // NCHUNK)
        c = g % NCHUNK
        return pltpu.make_async_copy(
            inp.at[s, pl.ds(c * CHUNK, CHUNK)], buf, sem)

    chunk_copy(0, bufs[0], sems[0]).start()

    def combine(a, b):
        av, ai = a
        bv, bi = b
        better = (bv > av) | ((bv == av) & (bi < ai))
        return jnp.where(better, bv, av), jnp.where(better, bi, ai)

    idx_lanes = jnp.zeros((LANES,), jnp.int32)
    total = SPW * NCHUNK
    neg_inf = jnp.full((LANES,), -jnp.inf, jnp.float32)
    zeros_i = jnp.zeros((LANES,), jnp.int32)
    for s_local in range(SPW):
        run_max = neg_inf
        run_idx = zeros_i
        for c in range(NCHUNK):
            g = s_local * NCHUNK + c
            buf, sem = bufs[g % 2], sems[g % 2]
            if g + 1 < total:
                chunk_copy(g + 1, bufs[(g + 1) % 2], sems[(g + 1) % 2]).start()
            chunk_copy(g, buf, sem).wait()

            def body(i, carry, buf=buf):
                i_vec = jnp.full((LANES,), i, jnp.int32)
                out = []
                for k in range(UNROLL):
                    rm, ri = carry[2 * k], carry[2 * k + 1]
                    v = buf[pl.ds(i * (UNROLL * LANES) + k * LANES, LANES)]
                    m = v > rm
                    out.append(jnp.where(m, v, rm))
                    out.append(jnp.where(m, i_vec, ri))
                return tuple(out)

            init = (neg_inf, zeros_i) * UNROLL
            acc = lax.fori_loop(0, CHUNK // (UNROLL * LANES), body, init)
            pairs = []
            for k in range(UNROLL):
                rm, ri = acc[2 * k], acc[2 * k + 1]
                fi = ri * (UNROLL * LANES) + (c * CHUNK + k * LANES) + iota
                pairs.append((rm, fi))
            while len(pairs) > 1:
                pairs = [combine(pairs[j], pairs[j + 1])
                         for j in range(0, len(pairs), 2)]
            run_max, run_idx = combine((run_max, run_idx), pairs[0])
        m = _rotreduce(run_max, tmpf, jnp.maximum)
        cand = jnp.where(run_max == m, run_idx, jnp.int32(INT_MAX))
        best = _rotreduce(cand, tmpi, jnp.minimum)
        idx_lanes = jnp.where(iota == s_local, best, idx_lanes)

    idxv[...] = idx_lanes
    pltpu.make_async_copy(gx.at[idxv], gatv, gsem).start()
    pltpu.make_async_copy(gx.at[idxv], gatv, gsem).wait()
    pltpu.sync_copy(gatv, outx.at[wid])
    pltpu.make_async_copy(gy.at[idxv], gatv, gsem).start()
    pltpu.make_async_copy(gy.at[idxv], gatv, gsem).wait()
    pltpu.sync_copy(gatv, outy.at[wid])


@jax.jit
def kernel(input, grid_x, grid_y):
    inp2 = input.reshape(B, HW)
    gx1 = grid_x.reshape(HW)
    gy1 = grid_y.reshape(HW)
    call = functools.partial(
        pl.kernel,
        out_type=[
            jax.ShapeDtypeStruct((NW, LANES), jnp.float32),
            jax.ShapeDtypeStruct((NW, LANES), jnp.float32),
        ],
        mesh=plsc.VectorSubcoreMesh(core_axis_name="c", subcore_axis_name="s"),
        scratch_types=[
            pltpu.VMEM((CHUNK,), jnp.float32),
            pltpu.VMEM((CHUNK,), jnp.float32),
            pltpu.VMEM((LANES,), jnp.int32),
            pltpu.VMEM((LANES,), jnp.float32),
            pltpu.VMEM((2 * LANES,), jnp.float32),
            pltpu.VMEM((2 * LANES,), jnp.int32),
            pltpu.SemaphoreType.DMA,
            pltpu.SemaphoreType.DMA,
            pltpu.SemaphoreType.DMA,
        ],
    )(_argmax_gather_kernel)
    outx, outy = call(inp2, gx1, gy1)
    x = outx[:, :SPW].reshape(B, 1)
    y = outy[:, :SPW].reshape(B, 1)
    return jnp.concatenate((x, y), axis=1)
